# baseline (speedup 1.0000x reference)
import functools

import jax
import jax.numpy as jnp
from jax.experimental import pallas as pl
from jax.experimental.pallas import tpu as pltpu


def _round_up(v, m):
    return ((v + m - 1) // m) * m


def _ca_kernel(x_ref, wa_ref, ba_ref, ws_ref, bs_ref, o_ref, *, hw_real, bb, c):
    x = x_ref[...]
    s = jnp.sum(x, axis=1, keepdims=True)
    ss = jnp.sum(x * x, axis=1, keepdims=True)

    hw = jnp.float32(hw_real)
    denom = jnp.float32(max(hw_real - 1, 1))
    mean = s / hw
    var = jnp.maximum((ss - hw * mean * mean) / denom, 0.0)
    si = jnp.sqrt(var)

    wa = wa_ref[...]
    ws = ws_ref[...]
    ba = ba_ref[...]
    bs = bs_ref[...]

    gates = []
    for i in range(bb):
        m_i = mean[i * c:(i + 1) * c, :]
        s_i = si[i * c:(i + 1) * c, :]
        a = jnp.maximum(
            jax.lax.dot(wa, m_i, preferred_element_type=jnp.float32) + ba, 0.0)
        b = jnp.maximum(
            jax.lax.dot(ws, s_i, preferred_element_type=jnp.float32) + bs, 0.0)
        gates.append(jax.nn.sigmoid(a + b))
    gate = jnp.concatenate(gates, axis=0) if bb > 1 else gates[0]

    o_ref[...] = x * gate


def kernel(x, w_avg, b_avg, w_si, b_si):
    B, C, H, W = x.shape
    HW = H * W
    hw_pad = _round_up(HW, 128)

    xf = x.reshape(B * C, HW)
    if hw_pad != HW:
        xf = jnp.pad(xf, ((0, 0), (0, hw_pad - HW)))

    wa = jnp.asarray(w_avg)
    ws = jnp.asarray(w_si)
    ba = jnp.asarray(b_avg).reshape(C, 1)
    bs = jnp.asarray(b_si).reshape(C, 1)

    bb = 1
    itemsize = jnp.dtype(x.dtype).itemsize
    while bb < B and B % (bb * 2) == 0 and (bb * 2) * C * hw_pad * itemsize <= (2 << 20):
        bb *= 2

    out = pl.pallas_call(
        functools.partial(_ca_kernel, hw_real=HW, bb=bb, c=C),
        out_shape=jax.ShapeDtypeStruct((B * C, hw_pad), x.dtype),
        grid=(B // bb,),
        in_specs=[
            pl.BlockSpec((bb * C, hw_pad), lambda b: (b, 0)),
            pl.BlockSpec((C, C), lambda b: (0, 0)),
            pl.BlockSpec((C, 1), lambda b: (0, 0)),
            pl.BlockSpec((C, C), lambda b: (0, 0)),
            pl.BlockSpec((C, 1), lambda b: (0, 0)),
        ],
        out_specs=pl.BlockSpec((bb * C, hw_pad), lambda b: (b, 0)),
        compiler_params=pltpu.CompilerParams(
            dimension_semantics=("parallel",), vmem_limit_bytes=64 << 20),
    )(xf, wa, ba, ws, bs)

    if hw_pad != HW:
        out = out[:, :HW]
    return out.reshape(B, C, H, W)


# speedup vs baseline: 9.0838x; 9.0838x over previous
import functools

import jax
import jax.numpy as jnp
from jax.experimental import pallas as pl
from jax.experimental.pallas import tpu as pltpu


def _round_up(v, m):
    return ((v + m - 1) // m) * m


def _ca_kernel(x_ref, wa_ref, ba_ref, ws_ref, bs_ref, o_ref, *, hw_real):
    x = x_ref[...]
    s = jnp.sum(x, axis=1)
    ss = jnp.sum(x * x, axis=1)

    hw = jnp.float32(hw_real)
    denom = jnp.float32(max(hw_real - 1, 1))
    mean = s / hw
    var = jnp.maximum((ss - hw * mean * mean) / denom, 0.0)
    si = jnp.sqrt(var)

    dn = (((1,), (1,)), ((), ()))
    a = jnp.maximum(
        jax.lax.dot_general(mean, wa_ref[...], dn,
                            preferred_element_type=jnp.float32) + ba_ref[...],
        0.0)
    b = jnp.maximum(
        jax.lax.dot_general(si, ws_ref[...], dn,
                            preferred_element_type=jnp.float32) + bs_ref[...],
        0.0)
    gate = jax.nn.sigmoid(a + b)

    o_ref[...] = x * gate[:, None, :]


def kernel(x, w_avg, b_avg, w_si, b_si):
    B, C, H, W = x.shape
    HW = H * W
    hw_pad = _round_up(HW, 8)
    c_pad = _round_up(C, 128)

    xt = jnp.transpose(x, (0, 2, 3, 1)).reshape(B, HW, C)
    if hw_pad != HW or c_pad != C:
        xt = jnp.pad(xt, ((0, 0), (0, hw_pad - HW), (0, c_pad - C)))

    wa = jnp.asarray(w_avg)
    ws = jnp.asarray(w_si)
    ba = jnp.asarray(b_avg).reshape(1, C)
    bs = jnp.asarray(b_si).reshape(1, C)
    if c_pad != C:
        wa = jnp.pad(wa, ((0, c_pad - C), (0, c_pad - C)))
        ws = jnp.pad(ws, ((0, c_pad - C), (0, c_pad - C)))
        ba = jnp.pad(ba, ((0, 0), (0, c_pad - C)))
        bs = jnp.pad(bs, ((0, 0), (0, c_pad - C)))

    itemsize = jnp.dtype(x.dtype).itemsize
    bb = 1
    while bb < B and B % (bb * 2) == 0 and (bb * 2) * c_pad * hw_pad * itemsize <= (8 << 20):
        bb *= 2

    out = pl.pallas_call(
        functools.partial(_ca_kernel, hw_real=HW),
        out_shape=jax.ShapeDtypeStruct((B, hw_pad, c_pad), x.dtype),
        grid=(B // bb,),
        in_specs=[
            pl.BlockSpec((bb, hw_pad, c_pad), lambda b: (b, 0, 0)),
            pl.BlockSpec((c_pad, c_pad), lambda b: (0, 0)),
            pl.BlockSpec((1, c_pad), lambda b: (0, 0)),
            pl.BlockSpec((c_pad, c_pad), lambda b: (0, 0)),
            pl.BlockSpec((1, c_pad), lambda b: (0, 0)),
        ],
        out_specs=pl.BlockSpec((bb, hw_pad, c_pad), lambda b: (b, 0, 0)),
        compiler_params=pltpu.CompilerParams(
            dimension_semantics=("parallel",), vmem_limit_bytes=64 << 20),
    )(xt, wa, ba, ws, bs)

    if hw_pad != HW or c_pad != C:
        out = out[:, :HW, :C]
    return out.reshape(B, H, W, C).transpose(0, 3, 1, 2)
